# baseline (speedup 1.0000x reference)
import functools

import jax
import jax.numpy as jnp
import numpy as np
from jax.experimental import pallas as pl

H = 128
N_NODES = 10000
N_EDGES = 160000
N_TRIPLETS = 160000


def _lin(v, p):
    return v @ p["W"] + p["b"]


def _rbf(d, vmin, vmax, bins):
    centers = jnp.linspace(vmin, vmax, bins)
    lengthscale = np.diff(np.linspace(vmin, vmax, bins)).mean()
    gamma = 1.0 / lengthscale
    return jnp.exp(-gamma * (d[:, None] - centers) ** 2)


def _egg_conv(node_feats, edge_feats, i, j, p, n_nodes):
    e_src = _lin(node_feats, p["src_gate"])[:, i]
    e_dst = _lin(node_feats, p["dst_gate"])[:, j]
    y = e_src + e_dst + _lin(edge_feats, p["edge_gate"])
    sigma = jax.nn.sigmoid(y)
    bh = _lin(node_feats, p["dst_update"])[:, j]
    m = bh * sigma
    bs, _, f = m.shape
    sum_sigma_h = jnp.zeros((bs, n_nodes, f), m.dtype).at[:, i].add(m)
    sum_sigma = jnp.zeros((bs, n_nodes, f), sigma.dtype).at[:, i].add(sigma)
    h = sum_sigma_h / (sum_sigma + 1e-6)
    xn = _lin(node_feats, p["src_update"]) + h
    xn = jax.nn.silu(xn)
    y = jax.nn.silu(y)
    xn = node_feats + xn
    y = edge_feats + y
    return xn, y


def _readout_kernel(x_ref, w_ref, b_ref, o_ref):
    h = jnp.mean(x_ref[0], axis=0, keepdims=True)
    o_ref[...] = h @ w_ref[...] + b_ref[...]


def _readout(x, w, b):
    return pl.pallas_call(
        _readout_kernel,
        out_shape=jax.ShapeDtypeStruct((1, 1), jnp.float32),
    )(x, w, b)


def kernel(x, dist, angle, params, edge_i, edge_j, t_i, t_j):
    bs = x.shape[0]
    x = jax.nn.silu(_lin(x, params["atom"]))
    y = jax.nn.silu(_lin(jax.nn.silu(_lin(_rbf(dist, 0.0, 8.0, 80), params["edge_mlp1"])), params["edge_mlp2"]))
    y = jnp.tile(y[None], (bs, 1, 1))
    z = jax.nn.silu(_lin(jax.nn.silu(_lin(_rbf(angle, -1.0, 1.0, 40), params["angle_mlp1"])), params["angle_mlp2"]))
    z = jnp.tile(z[None], (bs, 1, 1))
    for lp in params["alignn"]:
        m, z = _egg_conv(y, z, t_i, t_j, lp["edge"], N_EDGES)
        x, y = _egg_conv(x, m, edge_i, edge_j, lp["node"], N_NODES)
    for lp in params["gcn"]:
        x, y = _egg_conv(x, y, edge_i, edge_j, lp, N_NODES)
    return _readout(x, params["out"]["W"], params["out"]["b"])


# speedup vs baseline: 1.2968x; 1.2968x over previous
import functools

import jax
import jax.numpy as jnp
import numpy as np
from jax import lax
from jax.experimental import pallas as pl
from jax.experimental.pallas import tpu as pltpu
from jax.experimental.pallas import tpu_sc as plsc

H = 128
N_NODES = 10000
N_EDGES = 160000
N_TRIPLETS = 160000

DB = 4096
PAD = 16
GB = 200
SB = 2000
GC = 128


def _silu(v):
    return v * jax.nn.sigmoid(v)


def _atom_kernel(x_ref, w_ref, b_ref, o_ref):
    o_ref[...] = _silu(x_ref[...] @ w_ref[...] + b_ref[0:1, :])


def _embed_atom(x, p):
    n = x.shape[0]
    bn = 400
    return pl.pallas_call(
        _atom_kernel,
        grid=(n // bn,),
        in_specs=[
            pl.BlockSpec((bn, 1), lambda i: (i, 0)),
            pl.BlockSpec((1, H), lambda i: (0, 0)),
            pl.BlockSpec((8, H), lambda i: (0, 0)),
        ],
        out_specs=pl.BlockSpec((bn, H), lambda i: (i, 0)),
        out_shape=jax.ShapeDtypeStruct((n, H), jnp.float32),
    )(x, p["W"], jnp.broadcast_to(p["b"], (8, H)))


def _rbf_kernel(d_ref, c_ref, w1_ref, b1_ref, w2_ref, b2_ref, o_ref, *, gamma):
    d = d_ref[...]
    r = jnp.exp(-gamma * (d - c_ref[0:1, :]) ** 2)
    h = _silu(r @ w1_ref[...] + b1_ref[0:1, :])
    o_ref[...] = _silu(h @ w2_ref[...] + b2_ref[0:1, :])


def _embed_rbf(d, p1, p2, vmin, vmax, bins):
    e = d.shape[0]
    be = 800
    mid = p1["W"].shape[1]
    centers = jnp.asarray(np.linspace(vmin, vmax, bins), jnp.float32)
    gamma = 1.0 / float(np.diff(np.linspace(vmin, vmax, bins)).mean())
    return pl.pallas_call(
        functools.partial(_rbf_kernel, gamma=gamma),
        grid=(e // be,),
        in_specs=[
            pl.BlockSpec((be, 1), lambda i: (i, 0)),
            pl.BlockSpec((8, bins), lambda i: (0, 0)),
            pl.BlockSpec((bins, mid), lambda i: (0, 0)),
            pl.BlockSpec((8, mid), lambda i: (0, 0)),
            pl.BlockSpec((mid, H), lambda i: (0, 0)),
            pl.BlockSpec((8, H), lambda i: (0, 0)),
        ],
        out_specs=pl.BlockSpec((be, H), lambda i: (i, 0)),
        out_shape=jax.ShapeDtypeStruct((e, H), jnp.float32),
    )(d[:, None], jnp.broadcast_to(centers, (8, bins)),
      p1["W"], jnp.broadcast_to(p1["b"], (8, mid)),
      p2["W"], jnp.broadcast_to(p2["b"], (8, H)))


def _mid_kernel(gi_ref, gj_ref, ef_ref, wg_ref, bg_ref, wdu_ref, bdu_ref,
                msig_ref, oef_ref):
    gi, gj, ef = gi_ref[...], gj_ref[...], ef_ref[...]
    cat = jnp.concatenate([gi, gj, ef], axis=1)
    yt = cat @ wg_ref[...] + bg_ref[0:1, :]
    sig = jax.nn.sigmoid(yt)
    m = (gj @ wdu_ref[...] + bdu_ref[0:1, :]) * sig
    msig_ref[...] = jnp.concatenate([m, sig], axis=1)
    oef_ref[...] = ef + yt * sig


def _mid(gi, gj, ef, wg, bg, wdu, bdu):
    e = gi.shape[0]
    be = 800
    return pl.pallas_call(
        _mid_kernel,
        grid=(e // be,),
        in_specs=[
            pl.BlockSpec((be, H), lambda i: (i, 0)),
            pl.BlockSpec((be, H), lambda i: (i, 0)),
            pl.BlockSpec((be, H), lambda i: (i, 0)),
            pl.BlockSpec((3 * H, H), lambda i: (0, 0)),
            pl.BlockSpec((8, H), lambda i: (0, 0)),
            pl.BlockSpec((H, H), lambda i: (0, 0)),
            pl.BlockSpec((8, H), lambda i: (0, 0)),
        ],
        out_specs=[
            pl.BlockSpec((be, 2 * H), lambda i: (i, 0)),
            pl.BlockSpec((be, H), lambda i: (i, 0)),
        ],
        out_shape=[
            jax.ShapeDtypeStruct((e, 2 * H), jnp.float32),
            jax.ShapeDtypeStruct((e, H), jnp.float32),
        ],
    )(gi, gj, ef, wg, bg, wdu, bdu)


def _post_kernel(*refs):
    nf_ref = refs[0]
    nd_refs = refs[1:-3]
    wsu_ref, bsu_ref, o_ref = refs[-3:]
    nf = nf_ref[...]
    nd = nd_refs[0][...]
    for r in nd_refs[1:]:
        nd = nd + r[...]
    h = nd[:, :H] / (nd[:, H:] + 1e-6)
    o_ref[...] = nf + _silu(nf @ wsu_ref[...] + bsu_ref[0:1, :] + h)


def _post(nf, nds, wsu, bsu):
    n = nf.shape[0]
    bn = 400 if n == N_NODES else 800
    return pl.pallas_call(
        _post_kernel,
        grid=(n // bn,),
        in_specs=[pl.BlockSpec((bn, H), lambda i: (i, 0))]
        + [pl.BlockSpec((bn, 2 * H), lambda i: (i, 0)) for _ in nds]
        + [
            pl.BlockSpec((H, H), lambda i: (0, 0)),
            pl.BlockSpec((8, H), lambda i: (0, 0)),
        ],
        out_specs=pl.BlockSpec((bn, H), lambda i: (i, 0)),
        out_shape=jax.ShapeDtypeStruct((n, H), jnp.float32),
    )(nf, *nds, wsu, jnp.broadcast_to(bsu, (8, H)))


def _readout_kernel(x_ref, w_ref, o_ref):
    h = jnp.mean(x_ref[...], axis=0, keepdims=True)
    o_ref[...] = h @ w_ref[...]


def _readout(nf, p):
    out = pl.pallas_call(
        _readout_kernel,
        out_shape=jax.ShapeDtypeStruct((1, 1), jnp.float32),
    )(nf, p["W"])
    return out + p["b"]


@functools.lru_cache(maxsize=None)
def _mesh():
    return plsc.VectorSubcoreMesh(core_axis_name="c", subcore_axis_name="s")


@functools.lru_cache(maxsize=None)
def _make_gather(rows, e, staged):
    per = e // 32
    nblk = per // GB
    scratch = [
        pltpu.VMEM((GB,), jnp.int32),
        pltpu.VMEM((GB, H), jnp.float32),
        pltpu.SemaphoreType.DMA,
    ]
    if staged:
        scratch.append(pltpu.VMEM_SHARED((rows, H), jnp.float32))

    @functools.partial(
        pl.kernel, mesh=_mesh(),
        out_type=[jax.ShapeDtypeStruct((e, H), jnp.float32)] * 2,
        scratch_types=scratch,
    )
    def k(table, ia, ib, ga, gb, i_v, b_v, sem, *rest):
        c = lax.axis_index("c")
        s = lax.axis_index("s")
        wid = s * 2 + c
        if staged:
            tbl = rest[0]

            @pl.when(s == 0)
            def _():
                pltpu.sync_copy(table, tbl)

            plsc.subcore_barrier()
            src = tbl
        else:
            src = table

        def body(blk, carry):
            base = pl.multiple_of(wid * per + blk * GB, 8)
            pltpu.sync_copy(ia.at[pl.ds(base, GB)], i_v)
            pltpu.async_copy(src.at[i_v], b_v, sem).wait()
            pltpu.sync_copy(b_v, ga.at[pl.ds(base, GB)])
            pltpu.sync_copy(ib.at[pl.ds(base, GB)], i_v)
            pltpu.async_copy(src.at[i_v], b_v, sem).wait()
            pltpu.sync_copy(b_v, gb.at[pl.ds(base, GB)])
            return carry

        lax.fori_loop(0, nblk, body, 0)

    return k


@functools.lru_cache(maxsize=None)
def _make_scatter(e, n, narr):
    k2 = narr // 2
    per = e // narr
    SCB = 128
    nfull = per // SCB
    npairs = nfull // 2
    tail = per - nfull * SCB
    nz1 = n // 112
    ztail = (n - nz1 * 112) // 16

    @functools.partial(
        pl.kernel, mesh=_mesh(),
        out_type=[jax.ShapeDtypeStruct((n, 2 * H), jnp.float32)] * narr,
        scratch_types=[
            pltpu.VMEM((SCB,), jnp.int32),
            pltpu.VMEM((SCB,), jnp.int32),
            pltpu.VMEM((SCB, 2 * H), jnp.float32),
            pltpu.VMEM((SCB, 2 * H), jnp.float32),
            pltpu.VMEM((112, 2 * H), jnp.float32),
            pltpu.VMEM((max(tail, 8),), jnp.int32),
            pltpu.VMEM((max(tail, 8), 2 * H), jnp.float32),
            pltpu.SemaphoreType.DMA,
            pltpu.SemaphoreType.DMA,
        ],
    )
    def k(msig, idx, *outs_scratch):
        outs = outs_scratch[:narr]
        (db0, db1, vb0, vb1, zbuf, dbt, vbt,
         sa0, sa1) = outs_scratch[narr:]
        c = lax.axis_index("c")
        s = lax.axis_index("s")

        def zf(i, carry):
            def zf2(jj, carry2):
                zbuf[i, pl.ds(jj * 16, 16)] = jnp.zeros((16,), jnp.float32)
                return carry2
            return lax.fori_loop(0, 2 * H // 16, zf2, carry)
        lax.fori_loop(0, 112, zf, 0)

        for a in range(narr):
            out = outs[a]

            @pl.when(c == (0 if a < k2 else 1))
            def _(out=out):
                def zc(q, carry):
                    row = pl.multiple_of((s + 16 * q) * 112, 8)
                    pltpu.sync_copy(zbuf, out.at[pl.ds(row, 112)])
                    return carry
                lax.fori_loop(0, (nz1 - s + 15) // 16, zc, 0)

                @pl.when(s == 0)
                def _():
                    for t in range(ztail):
                        pltpu.sync_copy(
                            zbuf.at[pl.ds(0, 16)],
                            out.at[pl.ds(nz1 * 112 + 16 * t, 16)])

        plsc.subcore_barrier()

        for a in range(narr):
            out = outs[a]
            base0 = a * per

            @pl.when((c == (0 if a < k2 else 1)) & (s == a % k2))
            def _(out=out, base0=base0):
                def pair(qq, carry):
                    b0 = pl.multiple_of(base0 + qq * 2 * SCB, 8)
                    b1 = pl.multiple_of(base0 + qq * 2 * SCB + SCB, 8)
                    pltpu.sync_copy(idx.at[pl.ds(b0, SCB)], db0)
                    pltpu.sync_copy(msig.at[pl.ds(b0, SCB)], vb0)
                    a0 = pltpu.async_copy(vb0, out.at[db0], sa0, add=True)
                    pltpu.sync_copy(idx.at[pl.ds(b1, SCB)], db1)
                    pltpu.sync_copy(msig.at[pl.ds(b1, SCB)], vb1)
                    a0.wait()
                    pltpu.async_copy(vb1, out.at[db1], sa1, add=True).wait()
                    return carry

                lax.fori_loop(0, npairs, pair, 0)
                done = npairs * 2 * SCB
                if nfull % 2:
                    b0 = pl.multiple_of(base0 + done, 8)
                    pltpu.sync_copy(idx.at[pl.ds(b0, SCB)], db0)
                    pltpu.sync_copy(msig.at[pl.ds(b0, SCB)], vb0)
                    pltpu.sync_copy(vb0, out.at[db0], add=True)
                    done += SCB
                if tail:
                    b0 = pl.multiple_of(base0 + done, 8)
                    pltpu.sync_copy(idx.at[pl.ds(b0, tail)], dbt)
                    pltpu.sync_copy(msig.at[pl.ds(b0, tail)], vbt)
                    pltpu.sync_copy(vbt, out.at[dbt], add=True)

    return k


def _conv(nf, ef, i, j, p, n):
    e = i.shape[0]
    ga, gb = _make_gather(n, e, n == N_NODES)(nf, i, j)
    wg = jnp.concatenate([p["src_gate"]["W"], p["dst_gate"]["W"],
                          p["edge_gate"]["W"]], axis=0)
    bg = p["src_gate"]["b"] + p["dst_gate"]["b"] + p["edge_gate"]["b"]
    msig, oef = _mid(ga, gb, ef, wg, jnp.broadcast_to(bg, (8, H)),
                     p["dst_update"]["W"],
                     jnp.broadcast_to(p["dst_update"]["b"], (8, H)))
    narr = 8 if n == N_NODES else 4
    nds = _make_scatter(e, n, narr)(msig, i)
    onf = _post(nf, nds, p["src_update"]["W"], p["src_update"]["b"])
    return onf, oef


def kernel(x, dist, angle, params, edge_i, edge_j, t_i, t_j):
    p = params
    nf = _embed_atom(x[0], p["atom"])
    y = _embed_rbf(dist, p["edge_mlp1"], p["edge_mlp2"], 0.0, 8.0, 80)
    z = _embed_rbf(angle, p["angle_mlp1"], p["angle_mlp2"], -1.0, 1.0, 40)
    ei = edge_i.astype(jnp.int32)
    ej = edge_j.astype(jnp.int32)
    ti = t_i.astype(jnp.int32)
    tj = t_j.astype(jnp.int32)
    for lp in p["alignn"]:
        m, z = _conv(y, z, ti, tj, lp["edge"], N_TRIPLETS)
        nf, y = _conv(nf, m, ei, ej, lp["node"], N_NODES)
    for lp in p["gcn"]:
        nf, y = _conv(nf, y, ei, ej, lp, N_NODES)
    return _readout(nf, p["out"])


# speedup vs baseline: 1.4444x; 1.1138x over previous
import functools

import jax
import jax.numpy as jnp
import numpy as np
from jax import lax
from jax.experimental import pallas as pl
from jax.experimental.pallas import tpu as pltpu
from jax.experimental.pallas import tpu_sc as plsc

H = 128
N_NODES = 10000
N_EDGES = 160000
N_TRIPLETS = 160000

DB = 4096
PAD = 16
GB = 200
SB = 2000
GC = 128


def _silu(v):
    return v * jax.nn.sigmoid(v)


def _atom_kernel(x_ref, w_ref, b_ref, o_ref):
    o_ref[...] = _silu(x_ref[...] @ w_ref[...] + b_ref[0:1, :])


def _embed_atom(x, p):
    n = x.shape[0]
    bn = 400
    return pl.pallas_call(
        _atom_kernel,
        grid=(n // bn,),
        in_specs=[
            pl.BlockSpec((bn, 1), lambda i: (i, 0)),
            pl.BlockSpec((1, H), lambda i: (0, 0)),
            pl.BlockSpec((8, H), lambda i: (0, 0)),
        ],
        out_specs=pl.BlockSpec((bn, H), lambda i: (i, 0)),
        out_shape=jax.ShapeDtypeStruct((n, H), jnp.float32),
    )(x, p["W"], jnp.broadcast_to(p["b"], (8, H)))


def _rbf_kernel(d_ref, c_ref, w1_ref, b1_ref, w2_ref, b2_ref, o_ref, *, gamma):
    d = d_ref[...]
    r = jnp.exp(-gamma * (d - c_ref[0:1, :]) ** 2)
    h = _silu(r @ w1_ref[...] + b1_ref[0:1, :])
    o_ref[...] = _silu(h @ w2_ref[...] + b2_ref[0:1, :])


def _embed_rbf(d, p1, p2, vmin, vmax, bins):
    e = d.shape[0]
    be = 800
    mid = p1["W"].shape[1]
    centers = jnp.asarray(np.linspace(vmin, vmax, bins), jnp.float32)
    gamma = 1.0 / float(np.diff(np.linspace(vmin, vmax, bins)).mean())
    return pl.pallas_call(
        functools.partial(_rbf_kernel, gamma=gamma),
        grid=(e // be,),
        in_specs=[
            pl.BlockSpec((be, 1), lambda i: (i, 0)),
            pl.BlockSpec((8, bins), lambda i: (0, 0)),
            pl.BlockSpec((bins, mid), lambda i: (0, 0)),
            pl.BlockSpec((8, mid), lambda i: (0, 0)),
            pl.BlockSpec((mid, H), lambda i: (0, 0)),
            pl.BlockSpec((8, H), lambda i: (0, 0)),
        ],
        out_specs=pl.BlockSpec((be, H), lambda i: (i, 0)),
        out_shape=jax.ShapeDtypeStruct((e, H), jnp.float32),
    )(d[:, None], jnp.broadcast_to(centers, (8, bins)),
      p1["W"], jnp.broadcast_to(p1["b"], (8, mid)),
      p2["W"], jnp.broadcast_to(p2["b"], (8, H)))


def _mid_kernel(gi_ref, gj_ref, ef_ref, wg_ref, bg_ref, wdu_ref, bdu_ref,
                msig_ref, oef_ref):
    gi, gj, ef = gi_ref[...], gj_ref[...], ef_ref[...]
    cat = jnp.concatenate([gi, gj, ef], axis=1)
    yt = cat @ wg_ref[...] + bg_ref[0:1, :]
    sig = jax.nn.sigmoid(yt)
    m = (gj @ wdu_ref[...] + bdu_ref[0:1, :]) * sig
    msig_ref[...] = jnp.concatenate([m, sig], axis=1)
    oef_ref[...] = ef + yt * sig


def _mid(gi, gj, ef, wg, bg, wdu, bdu):
    e = gi.shape[0]
    be = 800
    return pl.pallas_call(
        _mid_kernel,
        grid=(e // be,),
        in_specs=[
            pl.BlockSpec((be, H), lambda i: (i, 0)),
            pl.BlockSpec((be, H), lambda i: (i, 0)),
            pl.BlockSpec((be, H), lambda i: (i, 0)),
            pl.BlockSpec((3 * H, H), lambda i: (0, 0)),
            pl.BlockSpec((8, H), lambda i: (0, 0)),
            pl.BlockSpec((H, H), lambda i: (0, 0)),
            pl.BlockSpec((8, H), lambda i: (0, 0)),
        ],
        out_specs=[
            pl.BlockSpec((be, 2 * H), lambda i: (i, 0)),
            pl.BlockSpec((be, H), lambda i: (i, 0)),
        ],
        out_shape=[
            jax.ShapeDtypeStruct((e, 2 * H), jnp.float32),
            jax.ShapeDtypeStruct((e, H), jnp.float32),
        ],
    )(gi, gj, ef, wg, bg, wdu, bdu)


def _post_kernel(*refs):
    nf_ref = refs[0]
    nd_refs = refs[1:-3]
    wsu_ref, bsu_ref, o_ref = refs[-3:]
    nf = nf_ref[...]
    nd = nd_refs[0][...]
    for r in nd_refs[1:]:
        nd = nd + r[...]
    h = nd[:, :H] / (nd[:, H:] + 1e-6)
    o_ref[...] = nf + _silu(nf @ wsu_ref[...] + bsu_ref[0:1, :] + h)


def _post(nf, nds, wsu, bsu):
    n = nf.shape[0]
    bn = 400 if n == N_NODES else 800
    return pl.pallas_call(
        _post_kernel,
        grid=(n // bn,),
        in_specs=[pl.BlockSpec((bn, H), lambda i: (i, 0))]
        + [pl.BlockSpec((bn, 2 * H), lambda i: (i, 0)) for _ in nds]
        + [
            pl.BlockSpec((H, H), lambda i: (0, 0)),
            pl.BlockSpec((8, H), lambda i: (0, 0)),
        ],
        out_specs=pl.BlockSpec((bn, H), lambda i: (i, 0)),
        out_shape=jax.ShapeDtypeStruct((n, H), jnp.float32),
    )(nf, *nds, wsu, jnp.broadcast_to(bsu, (8, H)))


def _readout_kernel(x_ref, w_ref, o_ref):
    h = jnp.mean(x_ref[...], axis=0, keepdims=True)
    o_ref[...] = h @ w_ref[...]


def _readout(nf, p):
    out = pl.pallas_call(
        _readout_kernel,
        out_shape=jax.ShapeDtypeStruct((1, 1), jnp.float32),
    )(nf, p["W"])
    return out + p["b"]


@functools.lru_cache(maxsize=None)
def _mesh():
    return plsc.VectorSubcoreMesh(core_axis_name="c", subcore_axis_name="s")


@functools.lru_cache(maxsize=None)
def _make_gather(rows, e, staged):
    per = e // 32
    gbk = 40 if staged else GB
    nblk = per // gbk
    scratch = [
        pltpu.VMEM((per,), jnp.int32),
        pltpu.VMEM((per,), jnp.int32),
        pltpu.VMEM((gbk, H), jnp.float32),
        pltpu.VMEM((gbk, H), jnp.float32),
        pltpu.SemaphoreType.DMA,
        pltpu.SemaphoreType.DMA,
        pltpu.SemaphoreType.DMA,
        pltpu.SemaphoreType.DMA,
    ]
    if staged:
        scratch.append(pltpu.VMEM_SHARED((rows, H), jnp.float32))

    @functools.partial(
        pl.kernel, mesh=_mesh(),
        out_type=[jax.ShapeDtypeStruct((e, H), jnp.float32)] * 2,
        scratch_types=scratch,
    )
    def k(table, ia, ib, ga, gb, iva, ivb, bv0, bv1, sg0, sg1, sw0, sw1,
          *rest):
        c = lax.axis_index("c")
        s = lax.axis_index("s")
        wid = s * 2 + c
        base = pl.multiple_of(wid * per, 8)
        if staged:
            tbl = rest[0]

            @pl.when(s == 0)
            def _():
                pltpu.sync_copy(table, tbl)

            plsc.subcore_barrier()
            src = tbl
        else:
            src = table
        pltpu.sync_copy(ia.at[pl.ds(base, per)], iva)
        pltpu.sync_copy(ib.at[pl.ds(base, per)], ivb)

        def body(blk, carry):
            off = pl.multiple_of(blk * gbk, 8)
            g0 = pltpu.async_copy(src.at[iva.at[pl.ds(off, gbk)]], bv0, sg0)
            g0.wait()
            w0 = pltpu.async_copy(bv0, ga.at[pl.ds(base + off, gbk)], sw0)
            g1 = pltpu.async_copy(src.at[ivb.at[pl.ds(off, gbk)]], bv1, sg1)
            g1.wait()
            w1 = pltpu.async_copy(bv1, gb.at[pl.ds(base + off, gbk)], sw1)
            w0.wait()
            w1.wait()
            return carry

        lax.fori_loop(0, nblk, body, 0)

    return k


@functools.lru_cache(maxsize=None)
def _make_scatter(e, n, narr):
    k2 = narr // 2
    per = e // narr
    SCB = 128
    nfull = per // SCB
    npairs = nfull // 2
    tail = per - nfull * SCB
    nz1 = n // 112
    ztail = (n - nz1 * 112) // 16

    @functools.partial(
        pl.kernel, mesh=_mesh(),
        out_type=[jax.ShapeDtypeStruct((n, 2 * H), jnp.float32)] * narr,
        scratch_types=[
            pltpu.VMEM((SCB,), jnp.int32),
            pltpu.VMEM((SCB,), jnp.int32),
            pltpu.VMEM((SCB, 2 * H), jnp.float32),
            pltpu.VMEM((SCB, 2 * H), jnp.float32),
            pltpu.VMEM((112, 2 * H), jnp.float32),
            pltpu.VMEM((max(tail, 8),), jnp.int32),
            pltpu.VMEM((max(tail, 8), 2 * H), jnp.float32),
            pltpu.SemaphoreType.DMA,
            pltpu.SemaphoreType.DMA,
        ],
    )
    def k(msig, idx, *outs_scratch):
        outs = outs_scratch[:narr]
        (db0, db1, vb0, vb1, zbuf, dbt, vbt,
         sa0, sa1) = outs_scratch[narr:]
        c = lax.axis_index("c")
        s = lax.axis_index("s")

        def zf(i, carry):
            def zf2(jj, carry2):
                zbuf[i, pl.ds(jj * 16, 16)] = jnp.zeros((16,), jnp.float32)
                return carry2
            return lax.fori_loop(0, 2 * H // 16, zf2, carry)
        lax.fori_loop(0, 112, zf, 0)

        for a in range(narr):
            out = outs[a]

            @pl.when(c == (0 if a < k2 else 1))
            def _(out=out):
                def zc(q, carry):
                    row = pl.multiple_of((s + 16 * q) * 112, 8)
                    pltpu.sync_copy(zbuf, out.at[pl.ds(row, 112)])
                    return carry
                lax.fori_loop(0, (nz1 - s + 15) // 16, zc, 0)

                @pl.when(s == 0)
                def _():
                    for t in range(ztail):
                        pltpu.sync_copy(
                            zbuf.at[pl.ds(0, 16)],
                            out.at[pl.ds(nz1 * 112 + 16 * t, 16)])

        plsc.subcore_barrier()

        for a in range(narr):
            out = outs[a]
            base0 = a * per

            @pl.when((c == (0 if a < k2 else 1)) & (s == a % k2))
            def _(out=out, base0=base0):
                def pair(qq, carry):
                    b0 = pl.multiple_of(base0 + qq * 2 * SCB, 8)
                    b1 = pl.multiple_of(base0 + qq * 2 * SCB + SCB, 8)
                    pltpu.sync_copy(idx.at[pl.ds(b0, SCB)], db0)
                    pltpu.sync_copy(msig.at[pl.ds(b0, SCB)], vb0)
                    a0 = pltpu.async_copy(vb0, out.at[db0], sa0, add=True)
                    pltpu.sync_copy(idx.at[pl.ds(b1, SCB)], db1)
                    pltpu.sync_copy(msig.at[pl.ds(b1, SCB)], vb1)
                    a0.wait()
                    pltpu.async_copy(vb1, out.at[db1], sa1, add=True).wait()
                    return carry

                lax.fori_loop(0, npairs, pair, 0)
                done = npairs * 2 * SCB
                if nfull % 2:
                    b0 = pl.multiple_of(base0 + done, 8)
                    pltpu.sync_copy(idx.at[pl.ds(b0, SCB)], db0)
                    pltpu.sync_copy(msig.at[pl.ds(b0, SCB)], vb0)
                    pltpu.sync_copy(vb0, out.at[db0], add=True)
                    done += SCB
                if tail:
                    b0 = pl.multiple_of(base0 + done, 8)
                    pltpu.sync_copy(idx.at[pl.ds(b0, tail)], dbt)
                    pltpu.sync_copy(msig.at[pl.ds(b0, tail)], vbt)
                    pltpu.sync_copy(vbt, out.at[dbt], add=True)

    return k


def _conv(nf, ef, i, j, p, n):
    e = i.shape[0]
    ga, gb = _make_gather(n, e, n == N_NODES)(nf, i, j)
    wg = jnp.concatenate([p["src_gate"]["W"], p["dst_gate"]["W"],
                          p["edge_gate"]["W"]], axis=0)
    bg = p["src_gate"]["b"] + p["dst_gate"]["b"] + p["edge_gate"]["b"]
    msig, oef = _mid(ga, gb, ef, wg, jnp.broadcast_to(bg, (8, H)),
                     p["dst_update"]["W"],
                     jnp.broadcast_to(p["dst_update"]["b"], (8, H)))
    narr = 16 if n == N_NODES else 4
    nds = _make_scatter(e, n, narr)(msig, i)
    onf = _post(nf, nds, p["src_update"]["W"], p["src_update"]["b"])
    return onf, oef


def kernel(x, dist, angle, params, edge_i, edge_j, t_i, t_j):
    p = params
    nf = _embed_atom(x[0], p["atom"])
    y = _embed_rbf(dist, p["edge_mlp1"], p["edge_mlp2"], 0.0, 8.0, 80)
    z = _embed_rbf(angle, p["angle_mlp1"], p["angle_mlp2"], -1.0, 1.0, 40)
    ei = edge_i.astype(jnp.int32)
    ej = edge_j.astype(jnp.int32)
    ti = t_i.astype(jnp.int32)
    tj = t_j.astype(jnp.int32)
    for lp in p["alignn"]:
        m, z = _conv(y, z, ti, tj, lp["edge"], N_TRIPLETS)
        nf, y = _conv(nf, m, ei, ej, lp["node"], N_NODES)
    for lp in p["gcn"]:
        nf, y = _conv(nf, y, ei, ej, lp, N_NODES)
    return _readout(nf, p["out"])
